# baseline (speedup 1.0000x reference)
import functools

import jax
import jax.numpy as jnp
from jax import lax
from jax.experimental import pallas as pl
from jax.experimental.pallas import tpu as pltpu
from jax.experimental.pallas import tpu_sc as plsc

N = 10000
E = 160000
D = 256
H = 256
NCLS = 2

NC = 2
NS = 16
HALF = 144
W2PAD = 16
RB = 1000
GRID = N // RB

K1 = 80
NB1 = (E // NS) // K1
K2 = 40
NB2 = (E // (NC * NS)) // K2
NCH = 5
RPT = N // NS


def _tc_a_body(x_ref, w1l_ref, w1r_ref, b1_ref, paug_ref, r_ref):
    xb = x_ref[...]
    p = lax.dot_general(xb, w1l_ref[...], (((1,), (1,)), ((), ())),
                        preferred_element_type=jnp.float32)
    r = lax.dot_general(xb, w1r_ref[...], (((1,), (1,)), ((), ())),
                        preferred_element_type=jnp.float32) + b1_ref[...]
    ones = jnp.ones((RB, 1), jnp.float32)
    zeros = jnp.zeros((RB, 2 * HALF - D - 1), jnp.float32)
    paug_ref[0] = p[:, :HALF]
    paug_ref[1] = jnp.concatenate([p[:, HALF:], ones, zeros], axis=1)
    r_ref[...] = r


_tc_a = pl.pallas_call(
    _tc_a_body,
    grid=(GRID,),
    in_specs=[
        pl.BlockSpec((RB, D), lambda i: (i, 0)),
        pl.BlockSpec((H, D), lambda i: (0, 0)),
        pl.BlockSpec((H, D), lambda i: (0, 0)),
        pl.BlockSpec((1, H), lambda i: (0, 0)),
    ],
    out_specs=[
        pl.BlockSpec((NC, RB, HALF), lambda i: (0, i, 0)),
        pl.BlockSpec((RB, H), lambda i: (i, 0)),
    ],
    out_shape=[
        jax.ShapeDtypeStruct((NC, N, HALF), jnp.float32),
        jax.ShapeDtypeStruct((N, H), jnp.float32),
    ],
)


@functools.lru_cache(maxsize=None)
def _make_seg_sum(table_rows, width, nb, k):
    cb = nb // NCH

    def body(table_ref, src_ref, dst_ref, out_ref,
             src_a, src_b, dst_a, dst_b, rows0, rows1, acc,
             semi_a, semi_b, semg0, semg1, sems0, sems1):
        c = lax.axis_index("c")
        s = lax.axis_index("s")

        nlane = width // 16

        def _z(i, carry):
            r = i // nlane
            j = i % nlane
            rows0[r, pl.ds(j * 16, 16)] = jnp.zeros((16,), jnp.float32)
            return carry

        lax.fori_loop(0, k * nlane, _z, 0)
        nfull = RPT // k
        rem = RPT - nfull * k
        for q in range(nfull):
            pltpu.sync_copy(rows0, acc.at[pl.ds(s * RPT + q * k, k)])
        if rem:
            pltpu.sync_copy(rows0.at[pl.ds(0, rem)],
                            acc.at[pl.ds(s * RPT + nfull * k, rem)])

        pltpu.sync_copy(src_ref.at[c, s, pl.ds(0, cb)], src_a)
        pltpu.sync_copy(dst_ref.at[c, s, pl.ds(0, cb)], dst_a)
        plsc.subcore_barrier()

        bufs = [(src_a, dst_a, semi_a), (src_b, dst_b, semi_b)]
        for ch in range(NCH):
            sbuf, dbuf, semi = bufs[ch % 2]
            sbuf_n, dbuf_n, semi_n = bufs[(ch + 1) % 2]
            if ch > 0:
                pltpu.make_async_copy(
                    src_ref.at[c, s, pl.ds(ch * cb, cb)], sbuf, semi).wait()
                pltpu.make_async_copy(
                    dst_ref.at[c, s, pl.ds(ch * cb, cb)], dbuf, semi).wait()
            if ch + 1 < NCH:
                pltpu.async_copy(
                    src_ref.at[c, s, pl.ds((ch + 1) * cb, cb)], sbuf_n, semi_n)
                pltpu.async_copy(
                    dst_ref.at[c, s, pl.ds((ch + 1) * cb, cb)], dbuf_n, semi_n)

            pltpu.async_copy(table_ref.at[sbuf.at[0]], rows0, semg0)
            pltpu.async_copy(table_ref.at[sbuf.at[1]], rows1, semg1)

            def _pair(i, carry, sbuf=sbuf, dbuf=dbuf):
                b = 2 * i
                pltpu.make_async_copy(
                    table_ref.at[sbuf.at[b]], rows0, semg0).wait()
                d0 = pltpu.async_copy(
                    rows0, acc.at[dbuf.at[b]], sems0, add=True)
                pltpu.make_async_copy(
                    table_ref.at[sbuf.at[b + 1]], rows1, semg1).wait()
                d1 = pltpu.async_copy(
                    rows1, acc.at[dbuf.at[b + 1]], sems1, add=True)
                d0.wait()

                @pl.when(b + 2 < cb)
                def _():
                    pltpu.async_copy(table_ref.at[sbuf.at[b + 2]], rows0, semg0)

                d1.wait()

                @pl.when(b + 3 < cb)
                def _():
                    pltpu.async_copy(table_ref.at[sbuf.at[b + 3]], rows1, semg1)

                return carry

            lax.fori_loop(0, cb // 2, _pair, 0)
            if cb % 2:
                pltpu.make_async_copy(
                    table_ref.at[sbuf.at[cb - 1]], rows0, semg0).wait()
                pltpu.sync_copy(rows0, acc.at[dbuf.at[cb - 1]], add=True)

        plsc.subcore_barrier()
        pltpu.sync_copy(acc.at[pl.ds(s * RPT, RPT)],
                        out_ref.at[c, pl.ds(s * RPT, RPT)])

    return pl.kernel(
        body,
        out_type=jax.ShapeDtypeStruct((NC, N, width), jnp.float32),
        mesh=plsc.VectorSubcoreMesh(core_axis_name="c", subcore_axis_name="s"),
        scratch_types=[
            pltpu.VMEM((cb, k), jnp.int32),
            pltpu.VMEM((cb, k), jnp.int32),
            pltpu.VMEM((cb, k), jnp.int32),
            pltpu.VMEM((cb, k), jnp.int32),
            pltpu.VMEM((k, width), jnp.float32),
            pltpu.VMEM((k, width), jnp.float32),
            pltpu.VMEM_SHARED((N, width), jnp.float32),
            pltpu.SemaphoreType.DMA,
            pltpu.SemaphoreType.DMA,
            pltpu.SemaphoreType.DMA,
            pltpu.SemaphoreType.DMA,
            pltpu.SemaphoreType.DMA,
            pltpu.SemaphoreType.DMA,
        ],
        compiler_params=pltpu.CompilerParams(use_tc_tiling_on_sc=False),
    )


def _tc_b_body(seg_ref, r_ref, w2l_ref, w2r_ref, b2_ref, qaug_ref, r2c_ref):
    seg_a = seg_ref[0]
    seg_b = seg_ref[1]
    sums = jnp.concatenate([seg_a, seg_b[:, :D - HALF]], axis=1)
    denom = jnp.maximum(seg_b[:, D - HALF:D - HALF + 1], 1.0)
    h = jnp.maximum(sums / denom + r_ref[...], 0.0)
    q = lax.dot_general(h, w2l_ref[...], (((1,), (1,)), ((), ())),
                        preferred_element_type=jnp.float32)
    r2 = lax.dot_general(h, w2r_ref[...], (((1,), (1,)), ((), ())),
                         preferred_element_type=jnp.float32) + b2_ref[...]
    qaug_ref[...] = q
    r2c_ref[...] = jnp.concatenate(
        [r2[:, :NCLS], denom, jnp.zeros((RB, W2PAD - NCLS - 1), jnp.float32)],
        axis=1)


_tc_b = pl.pallas_call(
    _tc_b_body,
    grid=(GRID,),
    in_specs=[
        pl.BlockSpec((NC, RB, HALF), lambda i: (0, i, 0)),
        pl.BlockSpec((RB, H), lambda i: (i, 0)),
        pl.BlockSpec((W2PAD, H), lambda i: (0, 0)),
        pl.BlockSpec((W2PAD, H), lambda i: (0, 0)),
        pl.BlockSpec((1, W2PAD), lambda i: (0, 0)),
    ],
    out_specs=[
        pl.BlockSpec((RB, W2PAD), lambda i: (i, 0)),
        pl.BlockSpec((RB, W2PAD), lambda i: (i, 0)),
    ],
    out_shape=[
        jax.ShapeDtypeStruct((N, W2PAD), jnp.float32),
        jax.ShapeDtypeStruct((N, W2PAD), jnp.float32),
    ],
)


def _tc_c_body(seg2_ref, r2c_ref, out_ref):
    s2 = seg2_ref[0] + seg2_ref[1]
    r2c = r2c_ref[...]
    out_ref[...] = s2[:, :NCLS] / r2c[:, NCLS:NCLS + 1] + r2c[:, :NCLS]


_tc_c = pl.pallas_call(
    _tc_c_body,
    grid=(GRID,),
    in_specs=[
        pl.BlockSpec((NC, RB, W2PAD), lambda i: (0, i, 0)),
        pl.BlockSpec((RB, W2PAD), lambda i: (i, 0)),
    ],
    out_specs=pl.BlockSpec((RB, NCLS), lambda i: (i, 0)),
    out_shape=jax.ShapeDtypeStruct((N, NCLS), jnp.float32),
)


def kernel(x, edge_index, W1l, b1, W1r, W2l, b2, W2r):
    src = edge_index[0].astype(jnp.int32)
    dst = edge_index[1].astype(jnp.int32)

    paug, r1 = _tc_a(x, W1l, W1r, b1.reshape(1, H))
    table1 = paug.reshape(NC * N, HALF)

    srcp1 = jnp.stack([src, src + N]).reshape(NC, NS, NB1, K1)
    dst1 = jnp.stack([dst, dst]).reshape(NC, NS, NB1, K1)
    seg1 = _make_seg_sum(NC * N, HALF, NB1, K1)(table1, srcp1, dst1)

    w2lp = jnp.zeros((W2PAD, H), jnp.float32).at[:NCLS].set(W2l)
    w2rp = jnp.zeros((W2PAD, H), jnp.float32).at[:NCLS].set(W2r)
    b2p = jnp.zeros((1, W2PAD), jnp.float32).at[0, :NCLS].set(b2)
    qaug, r2c = _tc_b(seg1, r1, w2lp, w2rp, b2p)

    src2 = src.reshape(NC, NS, NB2, K2)
    dst2 = dst.reshape(NC, NS, NB2, K2)
    seg2 = _make_seg_sum(N, W2PAD, NB2, K2)(qaug, src2, dst2)

    return _tc_c(seg2, r2c)


# speedup vs baseline: 1.1451x; 1.1451x over previous
import functools

import jax
import jax.numpy as jnp
from jax import lax
from jax.experimental import pallas as pl
from jax.experimental.pallas import tpu as pltpu
from jax.experimental.pallas import tpu_sc as plsc

N = 10000
E = 160000
D = 256
H = 256
NCLS = 2

NC = 2
NS = 16
HALF = 144
W2PAD = 16
RB = 1000
GRID = N // RB

K1 = 80
NB1 = (E // NS) // K1
K2 = 128
NB2 = 40
EPT2 = E // (NC * NS)
N2 = N + 16
NCH = 5


def _tc_a_body(x_ref, w_ref, mask_ref, out_ref):
    t = lax.dot_general(x_ref[...], w_ref[...], (((1,), (1,)), ((), ())),
                        preferred_element_type=jnp.float32)
    half = pl.program_id(0) // GRID
    m = jnp.where(half == 0, mask_ref[0:1, :], mask_ref[1:2, :])
    out_ref[...] = t + m


_tc_a = pl.pallas_call(
    _tc_a_body,
    grid=(NC * GRID,),
    in_specs=[
        pl.BlockSpec((RB, D), lambda j: (j % GRID, 0)),
        pl.BlockSpec((HALF, D), lambda j: (j // GRID, 0)),
        pl.BlockSpec((NC, HALF), lambda j: (0, 0)),
    ],
    out_specs=pl.BlockSpec((RB, HALF), lambda j: (j, 0)),
    out_shape=jax.ShapeDtypeStruct((NC * N, HALF), jnp.float32),
)


@functools.lru_cache(maxsize=None)
def _make_seg_sum(width, nb, k, shared_idx, src_stride, acc_rows):
    cb = nb // NCH
    rpt = acc_rows // NS

    def _idx(buf, b):
        return buf.at[b]

    def body(table_ref, src_ref, dst_ref, out_ref,
             src_a, src_b, dst_a, dst_b, rows0, rows1, acc,
             semi_a, semi_b, semg0, semg1):
        c = lax.axis_index("c")
        s = lax.axis_index("s")

        nlane = width // 16

        def _z(i, carry):
            r = i // nlane
            j = i % nlane
            rows0[r, pl.ds(j * 16, 16)] = jnp.zeros((16,), jnp.float32)
            return carry

        lax.fori_loop(0, k * nlane, _z, 0)
        nfull = rpt // k
        rem = rpt - nfull * k
        for q in range(nfull):
            pltpu.sync_copy(rows0, acc.at[pl.ds(s * rpt + q * k, k)])
        if rem:
            pltpu.sync_copy(rows0.at[pl.ds(0, rem)],
                            acc.at[pl.ds(s * rpt + nfull * k, rem)])

        def _src_chunk(ch):
            if shared_idx:
                return src_ref.at[s, pl.ds(ch * cb, cb)]
            return src_ref.at[c, s, pl.ds(ch * cb, cb)]

        def _dst_chunk(ch):
            if shared_idx:
                return dst_ref.at[s, pl.ds(ch * cb, cb)]
            return dst_ref.at[c, s, pl.ds(ch * cb, cb)]

        pltpu.sync_copy(_src_chunk(0), src_a)
        pltpu.sync_copy(_dst_chunk(0), dst_a)
        plsc.subcore_barrier()

        bufs = [(src_a, dst_a, semi_a), (src_b, dst_b, semi_b)]
        for ch in range(NCH):
            sbuf, dbuf, semi = bufs[ch % 2]
            sbuf_n, dbuf_n, semi_n = bufs[(ch + 1) % 2]
            if ch > 0:
                pltpu.make_async_copy(_src_chunk(ch), sbuf, semi).wait()
                pltpu.make_async_copy(_dst_chunk(ch), dbuf, semi).wait()
            if ch + 1 < NCH:
                pltpu.async_copy(_src_chunk(ch + 1), sbuf_n, semi_n)
                pltpu.async_copy(_dst_chunk(ch + 1), dbuf_n, semi_n)

            if shared_idx:
                coff = c * src_stride

                def _ofs(i, carry, sbuf=sbuf):
                    r = i // (k // 16)
                    j = i % (k // 16)
                    sbuf[r, pl.ds(j * 16, 16)] = (
                        sbuf[r, pl.ds(j * 16, 16)] + coff)
                    return carry

                lax.fori_loop(0, cb * (k // 16), _ofs, 0)

            pltpu.async_copy(table_ref.at[_idx(sbuf, 0)], rows0, semg0)

            def _blk(b, carry, sbuf=sbuf, dbuf=dbuf):
                @pl.when(jnp.logical_and(b + 1 < cb, (b + 1) % 2 == 0))
                def _():
                    pltpu.async_copy(
                        table_ref.at[_idx(sbuf, b + 1)], rows0, semg0)

                @pl.when(jnp.logical_and(b + 1 < cb, (b + 1) % 2 == 1))
                def _():
                    pltpu.async_copy(
                        table_ref.at[_idx(sbuf, b + 1)], rows1, semg1)

                @pl.when(b % 2 == 0)
                def _():
                    pltpu.make_async_copy(
                        table_ref.at[_idx(sbuf, b)], rows0, semg0).wait()
                    pltpu.sync_copy(rows0, acc.at[_idx(dbuf, b)], add=True)

                @pl.when(b % 2 == 1)
                def _():
                    pltpu.make_async_copy(
                        table_ref.at[_idx(sbuf, b)], rows1, semg1).wait()
                    pltpu.sync_copy(rows1, acc.at[_idx(dbuf, b)], add=True)

                return carry

            lax.fori_loop(0, cb, _blk, 0)

        plsc.subcore_barrier()
        pltpu.sync_copy(acc.at[pl.ds(s * rpt, rpt)],
                        out_ref.at[c, pl.ds(s * rpt, rpt)])

    return pl.kernel(
        body,
        out_type=jax.ShapeDtypeStruct((NC, acc_rows, width), jnp.float32),
        mesh=plsc.VectorSubcoreMesh(core_axis_name="c", subcore_axis_name="s"),
        scratch_types=[
            pltpu.VMEM((cb, k), jnp.int32),
            pltpu.VMEM((cb, k), jnp.int32),
            pltpu.VMEM((cb, k), jnp.int32),
            pltpu.VMEM((cb, k), jnp.int32),
            pltpu.VMEM((k, width), jnp.float32),
            pltpu.VMEM((k, width), jnp.float32),
            pltpu.VMEM_SHARED((acc_rows, width), jnp.float32),
            pltpu.SemaphoreType.DMA,
            pltpu.SemaphoreType.DMA,
            pltpu.SemaphoreType.DMA,
            pltpu.SemaphoreType.DMA,
        ],
        compiler_params=pltpu.CompilerParams(use_tc_tiling_on_sc=False),
    )


def _tc_b_body(seg_ref, x_ref, w1r_ref, b1_ref, w2l_ref, w2r_ref, b2_ref,
               qaug_ref, r2c_ref):
    seg_a = seg_ref[0]
    seg_b = seg_ref[1]
    sums = jnp.concatenate([seg_a, seg_b[:, :D - HALF]], axis=1)
    denom = jnp.maximum(seg_b[:, D - HALF:D - HALF + 1], 1.0)
    r = lax.dot_general(x_ref[...], w1r_ref[...], (((1,), (1,)), ((), ())),
                        preferred_element_type=jnp.float32) + b1_ref[...]
    h = jnp.maximum(sums / denom + r, 0.0)
    q = lax.dot_general(h, w2l_ref[...], (((1,), (1,)), ((), ())),
                        preferred_element_type=jnp.float32)
    r2 = lax.dot_general(h, w2r_ref[...], (((1,), (1,)), ((), ())),
                         preferred_element_type=jnp.float32) + b2_ref[...]
    qaug_ref[...] = q
    r2c_ref[...] = jnp.concatenate(
        [r2[:, :NCLS], denom, jnp.zeros((RB, W2PAD - NCLS - 1), jnp.float32)],
        axis=1)


_tc_b = pl.pallas_call(
    _tc_b_body,
    grid=(GRID,),
    in_specs=[
        pl.BlockSpec((NC, RB, HALF), lambda i: (0, i, 0)),
        pl.BlockSpec((RB, D), lambda i: (i, 0)),
        pl.BlockSpec((H, D), lambda i: (0, 0)),
        pl.BlockSpec((1, H), lambda i: (0, 0)),
        pl.BlockSpec((W2PAD, H), lambda i: (0, 0)),
        pl.BlockSpec((W2PAD, H), lambda i: (0, 0)),
        pl.BlockSpec((1, W2PAD), lambda i: (0, 0)),
    ],
    out_specs=[
        pl.BlockSpec((RB, W2PAD), lambda i: (i, 0)),
        pl.BlockSpec((RB, W2PAD), lambda i: (i, 0)),
    ],
    out_shape=[
        jax.ShapeDtypeStruct((N, W2PAD), jnp.float32),
        jax.ShapeDtypeStruct((N, W2PAD), jnp.float32),
    ],
)


def _tc_c_body(seg2_ref, r2c_ref, out_ref):
    s2 = seg2_ref[0] + seg2_ref[1]
    r2c = r2c_ref[...]
    out_ref[...] = s2[:, :NCLS] / r2c[:, NCLS:NCLS + 1] + r2c[:, :NCLS]


_tc_c = pl.pallas_call(
    _tc_c_body,
    grid=(GRID,),
    in_specs=[
        pl.BlockSpec((NC, RB, W2PAD), lambda i: (0, i, 0)),
        pl.BlockSpec((RB, W2PAD), lambda i: (i, 0)),
    ],
    out_specs=pl.BlockSpec((RB, NCLS), lambda i: (i, 0)),
    out_shape=jax.ShapeDtypeStruct((N, NCLS), jnp.float32),
)


def kernel(x, edge_index, W1l, b1, W1r, W2l, b2, W2r):
    src = edge_index[0].astype(jnp.int32)
    dst = edge_index[1].astype(jnp.int32)

    w1lp = jnp.concatenate(
        [W1l, jnp.zeros((NC * HALF - H, D), jnp.float32)], axis=0)
    maskc = jnp.zeros((NC, HALF), jnp.float32).at[1, D - HALF].set(1.0)
    table1 = _tc_a(x, w1lp, maskc)

    src3 = src.reshape(NS, NB1, K1)
    dst3 = dst.reshape(NS, NB1, K1)
    seg1 = _make_seg_sum(HALF, NB1, K1, True, N, N)(table1, src3, dst3)

    w2lp = jnp.zeros((W2PAD, H), jnp.float32).at[:NCLS].set(W2l)
    w2rp = jnp.zeros((W2PAD, H), jnp.float32).at[:NCLS].set(W2r)
    b2p = jnp.zeros((1, W2PAD), jnp.float32).at[0, :NCLS].set(b2)
    qaug, r2c = _tc_b(seg1, x, W1r, b1.reshape(1, H), w2lp, w2rp, b2p)

    padc = ((0, 0), (0, NB2 * K2 - EPT2))
    src2 = jnp.pad(src.reshape(NC * NS, EPT2), padc).reshape(NC, NS, NB2, K2)
    dst2 = jnp.pad(dst.reshape(NC * NS, EPT2), padc,
                   constant_values=N).reshape(NC, NS, NB2, K2)
    seg2 = _make_seg_sum(W2PAD, NB2, K2, False, 0, N2)(qaug, src2, dst2)

    return _tc_c(seg2, r2c)


# speedup vs baseline: 1.3563x; 1.1845x over previous
import functools

import jax
import jax.numpy as jnp
from jax import lax
from jax.experimental import pallas as pl
from jax.experimental.pallas import tpu as pltpu
from jax.experimental.pallas import tpu_sc as plsc

N = 10000
E = 160000
D = 256
H = 256
NCLS = 2

NC = 2
NS = 16
HALF = 128
CW = 16
NCNT = N + 16
W2PAD = 16
RB = 1000
GRID = N // RB

K1 = 80
NB1 = (E // NS) // K1
CNT_SPLIT = 63
K2 = 128
NB2 = 40
EPT2 = E // (NC * NS)
NCH = 5


def _tc_a_body(x_ref, w_ref, out_ref):
    out_ref[...] = lax.dot_general(
        x_ref[...], w_ref[...], (((1,), (1,)), ((), ())),
        preferred_element_type=jnp.float32)


_tc_a = pl.pallas_call(
    _tc_a_body,
    grid=(NC * GRID,),
    in_specs=[
        pl.BlockSpec((RB, D), lambda j: (j % GRID, 0)),
        pl.BlockSpec((HALF, D), lambda j: (j // GRID, 0)),
    ],
    out_specs=pl.BlockSpec((RB, HALF), lambda j: (j, 0)),
    out_shape=jax.ShapeDtypeStruct((NC * N, HALF), jnp.float32),
)


@functools.lru_cache(maxsize=None)
def _make_seg_sum(width, nb, k, shared_idx, src_stride, acc_rows, with_cnt):
    cb = nb // NCH
    rpt = acc_rows // NS
    crpt = NCNT // NS

    def body(table_ref, src_ref, dst_ref, *rest):
        if with_cnt:
            (out_ref, cnt_ref, src_a, src_b, dst_a, dst_b, rows0, rows1,
             ones, acc, acc_cnt, semi_a, semi_b, semg0, semg1) = rest
        else:
            (out_ref, src_a, src_b, dst_a, dst_b, rows0, rows1,
             acc, semi_a, semi_b, semg0, semg1) = rest
        c = lax.axis_index("c")
        s = lax.axis_index("s")

        nlane = width // 16

        def _z(i, carry):
            r = i // nlane
            j = i % nlane
            rows0[r, pl.ds(j * 16, 16)] = jnp.zeros((16,), jnp.float32)
            return carry

        lax.fori_loop(0, k * nlane, _z, 0)
        nfull = rpt // k
        rem = rpt - nfull * k
        for q in range(nfull):
            pltpu.sync_copy(rows0, acc.at[pl.ds(s * rpt + q * k, k)])
        if rem:
            pltpu.sync_copy(rows0.at[pl.ds(0, rem)],
                            acc.at[pl.ds(s * rpt + nfull * k, rem)])

        if with_cnt:
            def _zc(i, carry):
                ones[i, pl.ds(0, 16)] = jnp.zeros((16,), jnp.float32)
                return carry

            lax.fori_loop(0, k, _zc, 0)
            cfull = crpt // k
            crem = crpt - cfull * k
            for q in range(cfull):
                pltpu.sync_copy(ones, acc_cnt.at[pl.ds(s * crpt + q * k, k)])
            if crem:
                pltpu.sync_copy(
                    ones.at[pl.ds(0, crem)],
                    acc_cnt.at[pl.ds(s * crpt + cfull * k, crem)])

            def _o(i, carry):
                ones[i, pl.ds(0, 16)] = jnp.ones((16,), jnp.float32)
                return carry

            lax.fori_loop(0, k, _o, 0)

        def _src_chunk(ch):
            if shared_idx:
                return src_ref.at[s, pl.ds(ch * cb, cb)]
            return src_ref.at[c, s, pl.ds(ch * cb, cb)]

        def _dst_chunk(ch):
            if shared_idx:
                return dst_ref.at[s, pl.ds(ch * cb, cb)]
            return dst_ref.at[c, s, pl.ds(ch * cb, cb)]

        pltpu.sync_copy(_src_chunk(0), src_a)
        pltpu.sync_copy(_dst_chunk(0), dst_a)
        plsc.subcore_barrier()

        bufs = [(src_a, dst_a, semi_a), (src_b, dst_b, semi_b)]
        for ch in range(NCH):
            sbuf, dbuf, semi = bufs[ch % 2]
            sbuf_n, dbuf_n, semi_n = bufs[(ch + 1) % 2]
            if ch > 0:
                pltpu.make_async_copy(_src_chunk(ch), sbuf, semi).wait()
                pltpu.make_async_copy(_dst_chunk(ch), dbuf, semi).wait()
            if ch + 1 < NCH:
                pltpu.async_copy(_src_chunk(ch + 1), sbuf_n, semi_n)
                pltpu.async_copy(_dst_chunk(ch + 1), dbuf_n, semi_n)

            if shared_idx:
                coff = c * src_stride

                def _ofs(i, carry, sbuf=sbuf):
                    r = i // (k // 16)
                    j = i % (k // 16)
                    sbuf[r, pl.ds(j * 16, 16)] = (
                        sbuf[r, pl.ds(j * 16, 16)] + coff)
                    return carry

                lax.fori_loop(0, cb * (k // 16), _ofs, 0)

            pltpu.async_copy(table_ref.at[sbuf.at[0]], rows0, semg0)

            def _blk(b, carry, sbuf=sbuf, dbuf=dbuf, ch=ch):
                @pl.when(jnp.logical_and(b + 1 < cb, (b + 1) % 2 == 0))
                def _():
                    pltpu.async_copy(table_ref.at[sbuf.at[b + 1]], rows0,
                                     semg0)

                @pl.when(jnp.logical_and(b + 1 < cb, (b + 1) % 2 == 1))
                def _():
                    pltpu.async_copy(table_ref.at[sbuf.at[b + 1]], rows1,
                                     semg1)

                if with_cnt:
                    inlow = (ch * cb + b) < CNT_SPLIT
                    mine = jnp.where(c == 0, inlow, jnp.logical_not(inlow))

                    @pl.when(mine)
                    def _():
                        pltpu.sync_copy(ones, acc_cnt.at[dbuf.at[b]],
                                        add=True)

                @pl.when(b % 2 == 0)
                def _():
                    pltpu.make_async_copy(
                        table_ref.at[sbuf.at[b]], rows0, semg0).wait()
                    pltpu.sync_copy(rows0, acc.at[dbuf.at[b]], add=True)

                @pl.when(b % 2 == 1)
                def _():
                    pltpu.make_async_copy(
                        table_ref.at[sbuf.at[b]], rows1, semg1).wait()
                    pltpu.sync_copy(rows1, acc.at[dbuf.at[b]], add=True)

                return carry

            lax.fori_loop(0, cb, _blk, 0)

        plsc.subcore_barrier()
        pltpu.sync_copy(acc.at[pl.ds(s * rpt, rpt)],
                        out_ref.at[c, pl.ds(s * rpt, rpt)])
        if with_cnt:
            pltpu.sync_copy(acc_cnt.at[pl.ds(s * crpt, crpt)],
                            cnt_ref.at[c, pl.ds(s * crpt, crpt)])

    out_type = [jax.ShapeDtypeStruct((NC, acc_rows, width), jnp.float32)]
    scratch = [
        pltpu.VMEM((cb, k), jnp.int32),
        pltpu.VMEM((cb, k), jnp.int32),
        pltpu.VMEM((cb, k), jnp.int32),
        pltpu.VMEM((cb, k), jnp.int32),
        pltpu.VMEM((k, width), jnp.float32),
        pltpu.VMEM((k, width), jnp.float32),
    ]
    if with_cnt:
        out_type.append(jax.ShapeDtypeStruct((NC, NCNT, CW), jnp.float32))
        scratch.append(pltpu.VMEM((k, CW), jnp.float32))
    scratch.append(pltpu.VMEM_SHARED((acc_rows, width), jnp.float32))
    if with_cnt:
        scratch.append(pltpu.VMEM_SHARED((NCNT, CW), jnp.float32))
    scratch += [pltpu.SemaphoreType.DMA] * 4

    return pl.kernel(
        body,
        out_type=out_type,
        mesh=plsc.VectorSubcoreMesh(core_axis_name="c", subcore_axis_name="s"),
        scratch_types=scratch,
        compiler_params=pltpu.CompilerParams(use_tc_tiling_on_sc=False),
    )


def _tc_b_body(seg_ref, cnt_ref, x_ref, w1r_ref, b1_ref, w2l_ref, w2r_ref,
               b2_ref, qaug_ref, r2c_ref):
    sums = jnp.concatenate([seg_ref[0], seg_ref[1]], axis=1)
    denom = jnp.maximum(cnt_ref[0, :, :1] + cnt_ref[1, :, :1], 1.0)
    r = lax.dot_general(x_ref[...], w1r_ref[...], (((1,), (1,)), ((), ())),
                        preferred_element_type=jnp.float32) + b1_ref[...]
    h = jnp.maximum(sums / denom + r, 0.0)
    q = lax.dot_general(h, w2l_ref[...], (((1,), (1,)), ((), ())),
                        preferred_element_type=jnp.float32)
    r2 = lax.dot_general(h, w2r_ref[...], (((1,), (1,)), ((), ())),
                         preferred_element_type=jnp.float32) + b2_ref[...]
    qaug_ref[...] = q
    r2c_ref[...] = jnp.concatenate(
        [r2[:, :NCLS], denom, jnp.zeros((RB, W2PAD - NCLS - 1), jnp.float32)],
        axis=1)


_tc_b = pl.pallas_call(
    _tc_b_body,
    grid=(GRID,),
    in_specs=[
        pl.BlockSpec((NC, RB, HALF), lambda i: (0, i, 0)),
        pl.BlockSpec((NC, RB, CW), lambda i: (0, i, 0)),
        pl.BlockSpec((RB, D), lambda i: (i, 0)),
        pl.BlockSpec((H, D), lambda i: (0, 0)),
        pl.BlockSpec((1, H), lambda i: (0, 0)),
        pl.BlockSpec((W2PAD, H), lambda i: (0, 0)),
        pl.BlockSpec((W2PAD, H), lambda i: (0, 0)),
        pl.BlockSpec((1, W2PAD), lambda i: (0, 0)),
    ],
    out_specs=[
        pl.BlockSpec((RB, W2PAD), lambda i: (i, 0)),
        pl.BlockSpec((RB, W2PAD), lambda i: (i, 0)),
    ],
    out_shape=[
        jax.ShapeDtypeStruct((N, W2PAD), jnp.float32),
        jax.ShapeDtypeStruct((N, W2PAD), jnp.float32),
    ],
)


def _tc_c_body(seg2_ref, r2c_ref, out_ref):
    s2 = seg2_ref[0] + seg2_ref[1]
    r2c = r2c_ref[...]
    out_ref[...] = s2[:, :NCLS] / r2c[:, NCLS:NCLS + 1] + r2c[:, :NCLS]


_tc_c = pl.pallas_call(
    _tc_c_body,
    grid=(GRID,),
    in_specs=[
        pl.BlockSpec((NC, RB, W2PAD), lambda i: (0, i, 0)),
        pl.BlockSpec((RB, W2PAD), lambda i: (i, 0)),
    ],
    out_specs=pl.BlockSpec((RB, NCLS), lambda i: (i, 0)),
    out_shape=jax.ShapeDtypeStruct((N, NCLS), jnp.float32),
)


def kernel(x, edge_index, W1l, b1, W1r, W2l, b2, W2r):
    src = edge_index[0].astype(jnp.int32)
    dst = edge_index[1].astype(jnp.int32)

    table1 = _tc_a(x, W1l)

    src3 = src.reshape(NS, NB1, K1)
    dst3 = dst.reshape(NS, NB1, K1)
    seg1, cnt1 = _make_seg_sum(HALF, NB1, K1, True, N, N, True)(
        table1, src3, dst3)

    w2lp = jnp.zeros((W2PAD, H), jnp.float32).at[:NCLS].set(W2l)
    w2rp = jnp.zeros((W2PAD, H), jnp.float32).at[:NCLS].set(W2r)
    b2p = jnp.zeros((1, W2PAD), jnp.float32).at[0, :NCLS].set(b2)
    qaug, r2c = _tc_b(seg1, cnt1, x, W1r, b1.reshape(1, H), w2lp, w2rp, b2p)

    padc = ((0, 0), (0, NB2 * K2 - EPT2))
    src2 = jnp.pad(src.reshape(NC * NS, EPT2), padc).reshape(NC, NS, NB2, K2)
    dst2 = jnp.pad(dst.reshape(NC * NS, EPT2), padc,
                   constant_values=N).reshape(NC, NS, NB2, K2)
    (seg2,) = _make_seg_sum(W2PAD, NB2, K2, False, 0, NCNT, False)(
        qaug, src2, dst2)

    return _tc_c(seg2, r2c)
